# Initial kernel scaffold; baseline (speedup 1.0000x reference)
import functools

import jax
import jax.numpy as jnp
from jax import lax
from jax.experimental import pallas as pl
from jax.experimental.pallas import tpu as pltpu
from jax.experimental.pallas import tpu_sc as plsc

N = 10000
D = 128
H = 128
C = 40
CP = 64
E = 320000

NC = 2
NS = 16
NW = NC * NS
EPW = E // NW
K = 80
NCHUNK = EPW // K
ZR = 25
RPW = N // NS
ND = 10240

_mesh = functools.partial(
    plsc.VectorSubcoreMesh,
    core_axis_name="c", subcore_axis_name="s", num_cores=NC, num_subcores=NS,
)


def _zero_rows(zbuf, nrows, width):
    zeros = jnp.zeros((16,), jnp.float32)

    def body(i, carry):
        for j in range(width // 16):
            zbuf[i, pl.ds(j * 16, 16)] = zeros
        return carry

    lax.fori_loop(0, nrows, body, 0)


def _make_prop(F):
    @functools.partial(
        pl.kernel,
        out_type=jax.ShapeDtypeStruct((NC, N, F), jnp.float32),
        mesh=_mesh(),
        scratch_types=[
            pltpu.VMEM((K,), jnp.int32),
            pltpu.VMEM((K,), jnp.int32),
            pltpu.VMEM((K, F), jnp.float32),
            pltpu.VMEM((ZR, F), jnp.float32),
            pltpu.VMEM_SHARED((N, F), jnp.float32),
            pltpu.SemaphoreType.DMA,
        ],
    )
    def prop(y_hbm, src_hbm, dst_hbm, out_hbm, src_v, dst_v, rows_v, zero_v,
             acc_sh, sem):
        c = lax.axis_index("c")
        s = lax.axis_index("s")
        wid = c * NS + s

        _zero_rows(zero_v, ZR, F)
        row0 = s * RPW

        def zcopy(j, carry):
            pltpu.sync_copy(zero_v, acc_sh.at[pl.ds(row0 + j * ZR, ZR)])
            return carry

        lax.fori_loop(0, RPW // ZR, zcopy, 0)
        plsc.subcore_barrier()

        ebase = wid * EPW

        def chunk(i, carry):
            base = pl.multiple_of(ebase + i * K, 8)
            pltpu.sync_copy(src_hbm.at[pl.ds(base, K)], src_v)
            pltpu.sync_copy(dst_hbm.at[pl.ds(base, K)], dst_v)
            pltpu.async_copy(y_hbm.at[src_v], rows_v, sem).wait()
            pltpu.sync_copy(rows_v, acc_sh.at[dst_v], add=True)
            return carry

        lax.fori_loop(0, NCHUNK, chunk, 0)
        plsc.subcore_barrier()

        pltpu.sync_copy(acc_sh.at[pl.ds(row0, RPW)],
                        out_hbm.at[c, pl.ds(row0, RPW)])

    return prop


def _make_deg():
    @functools.partial(
        pl.kernel,
        out_type=jax.ShapeDtypeStruct((NC, ND), jnp.float32),
        mesh=_mesh(),
        scratch_types=[
            pltpu.VMEM((K,), jnp.int32),
            pltpu.VMEM((K,), jnp.float32),
            pltpu.VMEM((ND // NS,), jnp.float32),
            pltpu.VMEM_SHARED((ND,), jnp.float32),
            pltpu.SemaphoreType.DMA,
        ],
    )
    def deg(dst_hbm, out_hbm, dst_v, ones_v, zero_v, acc_sh, sem):
        c = lax.axis_index("c")
        s = lax.axis_index("s")
        wid = c * NS + s
        zeros = jnp.zeros((16,), jnp.float32)
        ones = jnp.ones((16,), jnp.float32)

        def zbody(i, carry):
            zero_v[pl.ds(i * 16, 16)] = zeros
            return carry

        lax.fori_loop(0, (ND // NS) // 16, zbody, 0)
        for j in range(K // 16):
            ones_v[pl.ds(j * 16, 16)] = ones

        seg = ND // NS
        pltpu.sync_copy(zero_v, acc_sh.at[pl.ds(s * seg, seg)])
        plsc.subcore_barrier()

        ebase = wid * EPW

        def chunk(i, carry):
            base = pl.multiple_of(ebase + i * K, 8)
            pltpu.sync_copy(dst_hbm.at[pl.ds(base, K)], dst_v)
            pltpu.sync_copy(ones_v, acc_sh.at[dst_v], add=True)
            return carry

        lax.fori_loop(0, NCHUNK, chunk, 0)
        plsc.subcore_barrier()

        pltpu.sync_copy(acc_sh.at[pl.ds(s * seg, seg)],
                        out_hbm.at[c, pl.ds(s * seg, seg)])

    return deg


_prop_h = _make_prop(H)
_prop_c = _make_prop(CP)
_deg = _make_deg()

BN = 1250


def _tc_call(body, out_shapes, in_specs, out_specs):
    return pl.pallas_call(
        body,
        grid=(N // BN,),
        out_shape=out_shapes,
        in_specs=in_specs,
        out_specs=out_specs,
    )


def _rows_spec(width):
    return pl.BlockSpec((BN, width), lambda i: (i, 0))


def _acc_spec(width):
    return pl.BlockSpec((NC, BN, width), lambda i: (0, i, 0))


def _full_spec(a, b):
    return pl.BlockSpec((a, b), lambda i: (0, 0))


def _prologue_body(degp_ref, x_ref, w_ref, dinv_ref, y_ref):
    deg = degp_ref[:, 0:1] + degp_ref[:, 1:2] + 1.0
    dinv = lax.rsqrt(deg)
    dinv_ref[...] = dinv
    y_ref[...] = dinv * jnp.dot(x_ref[...], w_ref[...],
                                preferred_element_type=jnp.float32)


def _mid_body(acc_ref, y_ref, dinv_ref, b_ref, w_ref, h_ref, ynext_ref):
    dinv = dinv_ref[...]
    h = jnp.maximum(dinv * (acc_ref[0] + acc_ref[1] + y_ref[...]) + b_ref[...],
                    0.0)
    h_ref[...] = h
    ynext_ref[...] = dinv * jnp.dot(h, w_ref[...],
                                    preferred_element_type=jnp.float32)


def _proj_body(acc_ref, y_ref, dinv_ref, b_ref, x_ref, h1_ref, wo_ref,
               y3_ref):
    dinv = dinv_ref[...]
    h2 = jnp.maximum(dinv * (acc_ref[0] + acc_ref[1] + y_ref[...]) + b_ref[...],
                     0.0)
    z = jnp.dot(x_ref[...], wo_ref[0:D, :], preferred_element_type=jnp.float32)
    z += jnp.dot(h1_ref[...], wo_ref[D:D + H, :],
                 preferred_element_type=jnp.float32)
    z += jnp.dot(h2, wo_ref[D + H:, :], preferred_element_type=jnp.float32)
    y3_ref[...] = dinv * z


def _final_body(acc_ref, y_ref, dinv_ref, b_ref, out_ref):
    out_ref[...] = jnp.maximum(
        dinv_ref[...] * (acc_ref[0] + acc_ref[1] + y_ref[...]) + b_ref[...],
        0.0)


def kernel(x, edge_index, W1, b1, W2, b2, Wo, bo):
    src = edge_index[0]
    dst = edge_index[1]

    degp = _deg(dst)
    degp2 = degp[:, :N].T

    f32 = jnp.float32
    dinv, y1 = _tc_call(
        _prologue_body,
        (jax.ShapeDtypeStruct((N, 1), f32), jax.ShapeDtypeStruct((N, H), f32)),
        [_rows_spec(2), _rows_spec(D), _full_spec(D, H)],
        (_rows_spec(1), _rows_spec(H)),
    )(degp2, x, W1)

    acc1 = _prop_h(y1, src, dst)
    h1, y2 = _tc_call(
        _mid_body,
        (jax.ShapeDtypeStruct((N, H), f32), jax.ShapeDtypeStruct((N, H), f32)),
        [_acc_spec(H), _rows_spec(H), _rows_spec(1), _full_spec(1, H),
         _full_spec(H, H)],
        (_rows_spec(H), _rows_spec(H)),
    )(acc1, y1, dinv, b1.reshape(1, H), W2)

    acc2 = _prop_h(y2, src, dst)
    wo_pad = jnp.pad(Wo, ((0, 0), (0, CP - C)))
    bo_pad = jnp.pad(bo, (0, CP - C)).reshape(1, CP)
    y3 = _tc_call(
        _proj_body,
        jax.ShapeDtypeStruct((N, CP), f32),
        [_acc_spec(H), _rows_spec(H), _rows_spec(1), _full_spec(1, H),
         _rows_spec(D), _rows_spec(H), _full_spec(D + 2 * H, CP)],
        _rows_spec(CP),
    )(acc2, y2, dinv, b2.reshape(1, H), x, h1, wo_pad)

    acc3 = _prop_c(y3, src, dst)
    out = _tc_call(
        _final_body,
        jax.ShapeDtypeStruct((N, CP), f32),
        [_acc_spec(CP), _rows_spec(CP), _rows_spec(1), _full_spec(1, CP)],
        _rows_spec(CP),
    )(acc3, y3, dinv, bo_pad)

    return out[:, :C]


# speedup vs baseline: 10.9165x; 10.9165x over previous
import functools

import jax
import jax.numpy as jnp
from jax import lax
from jax.experimental import pallas as pl
from jax.experimental.pallas import tpu as pltpu
from jax.experimental.pallas import tpu_sc as plsc

N = 10000
D = 128
H = 128
C = 40
CP = 128
E = 320000

NC = 2
NS = 16
NW = NC * NS
EPW = E // NW
K = 80
NCHUNK = EPW // K
NP = 10240
RPW = NP // NS
ZR = 40
ND = 16384

_mesh = functools.partial(
    plsc.VectorSubcoreMesh,
    core_axis_name="c", subcore_axis_name="s", num_cores=NC, num_subcores=NS,
)


def _zero_rows(zbuf, nrows, width):
    zeros = jnp.zeros((16,), jnp.float32)

    def body(i, carry):
        for j in range(width // 16):
            zbuf[i, pl.ds(j * 16, 16)] = zeros
        return carry

    lax.fori_loop(0, nrows, body, 0)


def _make_prop(F):
    @functools.partial(
        pl.kernel,
        out_type=jax.ShapeDtypeStruct((NC, NP, F), jnp.float32),
        mesh=_mesh(),
        scratch_types=[
            pltpu.VMEM((K,), jnp.int32),
            pltpu.VMEM((K,), jnp.int32),
            pltpu.VMEM((K, F), jnp.float32),
            pltpu.VMEM((ZR, F), jnp.float32),
            pltpu.VMEM_SHARED((NP, F), jnp.float32),
            pltpu.SemaphoreType.DMA,
        ],
    )
    def prop(y_hbm, src_hbm, dst_hbm, out_hbm, src_v, dst_v, rows_v, zero_v,
             acc_sh, sem):
        c = lax.axis_index("c")
        s = lax.axis_index("s")
        wid = c * NS + s

        _zero_rows(zero_v, ZR, F)
        row0 = s * RPW

        def zcopy(j, carry):
            pltpu.sync_copy(zero_v, acc_sh.at[pl.ds(row0 + j * ZR, ZR)])
            return carry

        lax.fori_loop(0, RPW // ZR, zcopy, 0)
        plsc.subcore_barrier()

        ebase = wid * EPW

        def chunk(i, carry):
            base = pl.multiple_of(ebase + i * K, 8)
            pltpu.sync_copy(src_hbm.at[pl.ds(base, K)], src_v)
            pltpu.sync_copy(dst_hbm.at[pl.ds(base, K)], dst_v)
            pltpu.async_copy(y_hbm.at[src_v], rows_v, sem).wait()
            pltpu.sync_copy(rows_v, acc_sh.at[dst_v], add=True)
            return carry

        lax.fori_loop(0, NCHUNK, chunk, 0)
        plsc.subcore_barrier()

        pltpu.sync_copy(acc_sh.at[pl.ds(row0, RPW)],
                        out_hbm.at[c, pl.ds(row0, RPW)])

    return prop


def _make_deg():
    @functools.partial(
        pl.kernel,
        out_type=jax.ShapeDtypeStruct((NC * ND,), jnp.float32),
        mesh=_mesh(),
        scratch_types=[
            pltpu.VMEM((K,), jnp.int32),
            pltpu.VMEM((K,), jnp.float32),
            pltpu.VMEM((ND // NS,), jnp.float32),
            pltpu.VMEM_SHARED((ND,), jnp.float32),
            pltpu.SemaphoreType.DMA,
        ],
    )
    def deg(dst_hbm, out_hbm, dst_v, ones_v, zero_v, acc_sh, sem):
        c = lax.axis_index("c")
        s = lax.axis_index("s")
        wid = c * NS + s
        zeros = jnp.zeros((16,), jnp.float32)
        ones = jnp.ones((16,), jnp.float32)

        def zbody(i, carry):
            zero_v[pl.ds(i * 16, 16)] = zeros
            return carry

        lax.fori_loop(0, (ND // NS) // 16, zbody, 0)
        for j in range(K // 16):
            ones_v[pl.ds(j * 16, 16)] = ones

        seg = ND // NS
        pltpu.sync_copy(zero_v, acc_sh.at[pl.ds(s * seg, seg)])
        plsc.subcore_barrier()

        ebase = wid * EPW

        def chunk(i, carry):
            base = pl.multiple_of(ebase + i * K, 8)
            pltpu.sync_copy(dst_hbm.at[pl.ds(base, K)], dst_v)
            pltpu.sync_copy(ones_v, acc_sh.at[dst_v], add=True)
            return carry

        lax.fori_loop(0, NCHUNK, chunk, 0)
        plsc.subcore_barrier()

        pltpu.sync_copy(acc_sh.at[pl.ds(s * seg, seg)],
                        out_hbm.at[pl.ds(c * ND + s * seg, seg)])

    return deg


_prop_h = _make_prop(H)
_prop_c = _make_prop(CP)
_deg = _make_deg()

BN = 2000


def _tc_call(body, out_shapes, in_specs, out_specs):
    return pl.pallas_call(
        body,
        grid=(N // BN,),
        out_shape=out_shapes,
        in_specs=in_specs,
        out_specs=out_specs,
    )


def _rows_spec(width):
    return pl.BlockSpec((BN, width), lambda i: (i, 0))


def _acc_spec(width):
    return pl.BlockSpec((NC, BN, width), lambda i: (0, i, 0))


def _full_spec(a, b):
    return pl.BlockSpec((a, b), lambda i: (0, 0))


def _prologue_body(degp_ref, x_ref, w_ref, dinv_ref, y_ref):
    deg = degp_ref[:, 0:1] + degp_ref[:, 1:2] + 1.0
    dinv = lax.rsqrt(deg)
    dinv_ref[...] = dinv
    y_ref[...] = dinv * jnp.dot(x_ref[...], w_ref[...],
                                preferred_element_type=jnp.float32)


def _mid_body(acc_ref, y_ref, dinv_ref, b_ref, w_ref, h_ref, ynext_ref):
    dinv = dinv_ref[...]
    h = jnp.maximum(dinv * (acc_ref[0] + acc_ref[1] + y_ref[...]) + b_ref[...],
                    0.0)
    h_ref[...] = h
    ynext_ref[...] = dinv * jnp.dot(h, w_ref[...],
                                    preferred_element_type=jnp.float32)


def _proj_body(acc_ref, y_ref, dinv_ref, b_ref, x_ref, h1_ref, wo_ref,
               y3_ref):
    dinv = dinv_ref[...]
    h2 = jnp.maximum(dinv * (acc_ref[0] + acc_ref[1] + y_ref[...]) + b_ref[...],
                     0.0)
    z = jnp.dot(x_ref[...], wo_ref[0:D, :], preferred_element_type=jnp.float32)
    z += jnp.dot(h1_ref[...], wo_ref[D:D + H, :],
                 preferred_element_type=jnp.float32)
    z += jnp.dot(h2, wo_ref[D + H:, :], preferred_element_type=jnp.float32)
    y3_ref[...] = dinv * z


def _final_body(acc_ref, y_ref, dinv_ref, b_ref, out_ref):
    out_ref[...] = jnp.maximum(
        dinv_ref[...] * (acc_ref[0] + acc_ref[1] + y_ref[...]) + b_ref[...],
        0.0)


def kernel(x, edge_index, W1, b1, W2, b2, Wo, bo):
    src = edge_index[0]
    dst = edge_index[1]

    degp = _deg(dst)
    degp2 = degp.reshape(NC, ND)[:, :N].T

    f32 = jnp.float32
    dinv, y1 = _tc_call(
        _prologue_body,
        (jax.ShapeDtypeStruct((N, 1), f32), jax.ShapeDtypeStruct((N, H), f32)),
        [_rows_spec(2), _rows_spec(D), _full_spec(D, H)],
        (_rows_spec(1), _rows_spec(H)),
    )(degp2, x, W1)

    acc1 = _prop_h(y1, src, dst)
    h1, y2 = _tc_call(
        _mid_body,
        (jax.ShapeDtypeStruct((N, H), f32), jax.ShapeDtypeStruct((N, H), f32)),
        [_acc_spec(H), _rows_spec(H), _rows_spec(1), _full_spec(1, H),
         _full_spec(H, H)],
        (_rows_spec(H), _rows_spec(H)),
    )(acc1, y1, dinv, b1.reshape(1, H), W2)

    acc2 = _prop_h(y2, src, dst)
    wo_pad = jnp.pad(Wo, ((0, 0), (0, CP - C)))
    bo_pad = jnp.pad(bo, (0, CP - C)).reshape(1, CP)
    y3 = _tc_call(
        _proj_body,
        jax.ShapeDtypeStruct((N, CP), f32),
        [_acc_spec(H), _rows_spec(H), _rows_spec(1), _full_spec(1, H),
         _rows_spec(D), _rows_spec(H), _full_spec(D + 2 * H, CP)],
        _rows_spec(CP),
    )(acc2, y2, dinv, b2.reshape(1, H), x, h1, wo_pad)

    acc3 = _prop_c(y3, src, dst)
    out = _tc_call(
        _final_body,
        jax.ShapeDtypeStruct((N, CP), f32),
        [_acc_spec(CP), _rows_spec(CP), _rows_spec(1), _full_spec(1, CP)],
        _rows_spec(CP),
    )(acc3, y3, dinv, bo_pad)

    return out[:, :C]
